# Initial kernel scaffold; baseline (speedup 1.0000x reference)
import functools

import jax
import jax.numpy as jnp
from jax import lax
from jax.experimental import pallas as pl
from jax.experimental.pallas import tpu as pltpu
from jax.experimental.pallas import tpu_sc as plsc

N = 10000
E = 320000
D_IN = 128
D = 16

NC = 2
NS = 16
NW = NC * NS
EPW = E // NW
K = 2000
NCH = EPW // K
NPT = N // NS

RB = 2000


def _proj1_body(x_ref, wl_ref, wr_ref, b_ref, y_ref, r_ref):
    xb = x_ref[...]
    y_ref[...] = jnp.dot(xb, wl_ref[...], preferred_element_type=jnp.float32)
    r_ref[...] = (
        jnp.dot(xb, wr_ref[...], preferred_element_type=jnp.float32) + b_ref[...]
    )


def _proj1(x, W1l, W1r, b1):
    return pl.pallas_call(
        _proj1_body,
        grid=(N // RB,),
        in_specs=[
            pl.BlockSpec((RB, D_IN), lambda i: (i, 0)),
            pl.BlockSpec((D_IN, D), lambda i: (0, 0)),
            pl.BlockSpec((D_IN, D), lambda i: (0, 0)),
            pl.BlockSpec((1, D), lambda i: (0, 0)),
        ],
        out_specs=[
            pl.BlockSpec((RB, D), lambda i: (i, 0)),
            pl.BlockSpec((RB, D), lambda i: (i, 0)),
        ],
        out_shape=[
            jax.ShapeDtypeStruct((N, D), jnp.float32),
            jax.ShapeDtypeStruct((N, D), jnp.float32),
        ],
    )(x, W1l, W1r, b1.reshape(1, D))


def _make_agg(with_cnt):
    mesh = plsc.VectorSubcoreMesh(core_axis_name="c", subcore_axis_name="s")
    out_type = [jax.ShapeDtypeStruct((NC * N, D), jnp.float32)]
    if with_cnt:
        out_type.append(jax.ShapeDtypeStruct((NC * N, D), jnp.float32))
    scratch_types = [
        pltpu.VMEM((K,), jnp.int32),
        pltpu.VMEM((K,), jnp.int32),
        pltpu.VMEM((K, D), jnp.float32),
        pltpu.VMEM((K, D), jnp.float32),
        pltpu.VMEM_SHARED((N, D), jnp.float32),
    ]
    if with_cnt:
        scratch_types.append(pltpu.VMEM_SHARED((N, D), jnp.float32))
    scratch_types.append(pltpu.SemaphoreType.DMA)

    def body(y_hbm, src_hbm, dst_hbm, *rest):
        if with_cnt:
            (out_hbm, cnt_hbm, src_v, dst_v, rows_v, fill_v,
             acc_sh, cnt_sh, sem) = rest
        else:
            (out_hbm, src_v, dst_v, rows_v, fill_v, acc_sh, sem) = rest
        c = lax.axis_index("c")
        s = lax.axis_index("s")
        wid = c * NS + s

        def zfill(i, _):
            fill_v[i, :] = jnp.zeros((D,), jnp.float32)
            return 0
        lax.fori_loop(0, NPT, zfill, 0)
        pltpu.sync_copy(fill_v.at[pl.ds(0, NPT)], acc_sh.at[pl.ds(s * NPT, NPT)])
        if with_cnt:
            pltpu.sync_copy(fill_v.at[pl.ds(0, NPT)],
                            cnt_sh.at[pl.ds(s * NPT, NPT)])

            def ofill(i, _):
                fill_v[i, :] = jnp.ones((D,), jnp.float32)
                return 0
            lax.fori_loop(0, K, ofill, 0)
        plsc.subcore_barrier()

        ebase = wid * EPW
        for j in range(NCH):
            off = ebase + j * K
            pltpu.sync_copy(src_hbm.at[pl.ds(off, K)], src_v)
            pltpu.sync_copy(dst_hbm.at[pl.ds(off, K)], dst_v)
            pltpu.async_copy(y_hbm.at[src_v], rows_v, sem).wait()
            pltpu.sync_copy(rows_v, acc_sh.at[dst_v], add=True)
            if with_cnt:
                pltpu.sync_copy(fill_v, cnt_sh.at[dst_v], add=True)
        plsc.subcore_barrier()

        rb = s * NPT
        ob = c * N + rb
        pltpu.sync_copy(acc_sh.at[pl.ds(rb, NPT)], out_hbm.at[pl.ds(ob, NPT)])
        if with_cnt:
            pltpu.sync_copy(cnt_sh.at[pl.ds(rb, NPT)],
                            cnt_hbm.at[pl.ds(ob, NPT)])

    return functools.partial(
        pl.kernel, mesh=mesh, out_type=out_type, scratch_types=scratch_types
    )(body)


_agg_cnt = _make_agg(with_cnt=True)
_agg = _make_agg(with_cnt=False)


def _mid_body(p_ref, c_ref, r_ref, wl_ref, wr_ref, b_ref, y_ref, rr_ref):
    agg = p_ref[0] + p_ref[1]
    cnt = c_ref[0] + c_ref[1]
    inv = 1.0 / jnp.maximum(cnt, 1.0)
    h = jnp.maximum(agg * inv + r_ref[...], 0.0)
    y_ref[...] = jnp.dot(h, wl_ref[...], preferred_element_type=jnp.float32)
    rr_ref[...] = (
        jnp.dot(h, wr_ref[...], preferred_element_type=jnp.float32) + b_ref[...]
    )


def _mid(p1, c1, r1, W2l, W2r, b2):
    return pl.pallas_call(
        _mid_body,
        grid=(N // RB,),
        in_specs=[
            pl.BlockSpec((NC, RB, D), lambda i: (0, i, 0)),
            pl.BlockSpec((NC, RB, D), lambda i: (0, i, 0)),
            pl.BlockSpec((RB, D), lambda i: (i, 0)),
            pl.BlockSpec((D, D), lambda i: (0, 0)),
            pl.BlockSpec((D, D), lambda i: (0, 0)),
            pl.BlockSpec((1, D), lambda i: (0, 0)),
        ],
        out_specs=[
            pl.BlockSpec((RB, D), lambda i: (i, 0)),
            pl.BlockSpec((RB, D), lambda i: (i, 0)),
        ],
        out_shape=[
            jax.ShapeDtypeStruct((N, D), jnp.float32),
            jax.ShapeDtypeStruct((N, D), jnp.float32),
        ],
    )(p1.reshape(NC, N, D), c1.reshape(NC, N, D), r1, W2l, W2r, b2.reshape(1, D))


def _fin_body(p_ref, c_ref, r_ref, o_ref):
    agg = p_ref[0] + p_ref[1]
    cnt = c_ref[0] + c_ref[1]
    inv = 1.0 / jnp.maximum(cnt, 1.0)
    z = agg * inv + r_ref[...]
    m = jnp.max(z, axis=1, keepdims=True)
    e = jnp.exp(z - m)
    o_ref[...] = e / jnp.sum(e, axis=1, keepdims=True)


def _fin(p2, c1, r2):
    return pl.pallas_call(
        _fin_body,
        grid=(N // RB,),
        in_specs=[
            pl.BlockSpec((NC, RB, D), lambda i: (0, i, 0)),
            pl.BlockSpec((NC, RB, D), lambda i: (0, i, 0)),
            pl.BlockSpec((RB, D), lambda i: (i, 0)),
        ],
        out_specs=pl.BlockSpec((RB, D), lambda i: (i, 0)),
        out_shape=jax.ShapeDtypeStruct((N, D), jnp.float32),
    )(p2.reshape(NC, N, D), c1.reshape(NC, N, D), r2)


def kernel(x, edge_index, W1l, b1, W1r, W2l, b2, W2r):
    src = edge_index[0].astype(jnp.int32)
    dst = edge_index[1].astype(jnp.int32)
    y1, r1 = _proj1(x, W1l, W1r, b1)
    p1, c1 = _agg_cnt(y1, src, dst)
    y2, r2 = _mid(p1, c1, r1, W2l, W2r, b2)
    p2 = _agg(y2, src, dst)
    return _fin(p2, c1, r2)


# speedup vs baseline: 19.3370x; 19.3370x over previous
import functools

import jax
import jax.numpy as jnp
from jax import lax
from jax.experimental import pallas as pl
from jax.experimental.pallas import tpu as pltpu
from jax.experimental.pallas import tpu_sc as plsc

N = 10000
E = 320000
D_IN = 128
D = 16

NC = 2
NS = 16
NW = NC * NS
EPW = E // NW
K = 2000
NCH = EPW // K
NP = 10240
NPT = NP // NS

RB = 2000


def _proj1_body(x_ref, wl_ref, wr_ref, b_ref, y_ref, r_ref):
    xb = x_ref[...]
    y_ref[...] = jnp.dot(xb, wl_ref[...], preferred_element_type=jnp.float32)
    r_ref[...] = (
        jnp.dot(xb, wr_ref[...], preferred_element_type=jnp.float32) + b_ref[...]
    )


def _proj1(x, W1l, W1r, b1):
    return pl.pallas_call(
        _proj1_body,
        grid=(N // RB,),
        in_specs=[
            pl.BlockSpec((RB, D_IN), lambda i: (i, 0)),
            pl.BlockSpec((D_IN, D), lambda i: (0, 0)),
            pl.BlockSpec((D_IN, D), lambda i: (0, 0)),
            pl.BlockSpec((1, D), lambda i: (0, 0)),
        ],
        out_specs=[
            pl.BlockSpec((RB, D), lambda i: (i, 0)),
            pl.BlockSpec((RB, D), lambda i: (i, 0)),
        ],
        out_shape=[
            jax.ShapeDtypeStruct((N, D), jnp.float32),
            jax.ShapeDtypeStruct((N, D), jnp.float32),
        ],
    )(x, W1l, W1r, b1.reshape(1, D))


def _make_agg(with_cnt):
    mesh = plsc.VectorSubcoreMesh(core_axis_name="c", subcore_axis_name="s")
    out_type = [jax.ShapeDtypeStruct((NC * NP, D), jnp.float32)]
    if with_cnt:
        out_type.append(jax.ShapeDtypeStruct((NC * NP, D), jnp.float32))
    scratch_types = [
        pltpu.VMEM((K,), jnp.int32),
        pltpu.VMEM((K,), jnp.int32),
        pltpu.VMEM((K, D), jnp.float32),
        pltpu.VMEM((K, D), jnp.float32),
        pltpu.VMEM_SHARED((N, D), jnp.float32),
        pltpu.VMEM_SHARED((NP, D), jnp.float32),
    ]
    if with_cnt:
        scratch_types.append(pltpu.VMEM_SHARED((NP, D), jnp.float32))
    scratch_types.append(pltpu.SemaphoreType.DMA)

    def body(y_hbm, src_hbm, dst_hbm, *rest):
        if with_cnt:
            (out_hbm, cnt_hbm, src_v, dst_v, rows_v, fill_v,
             tab_sh, acc_sh, cnt_sh, sem) = rest
        else:
            (out_hbm, src_v, dst_v, rows_v, fill_v, tab_sh, acc_sh, sem) = rest
        c = lax.axis_index("c")
        s = lax.axis_index("s")
        wid = c * NS + s

        @pl.when(s < 10)
        def _stage():
            pltpu.sync_copy(y_hbm.at[pl.ds(s * 1000, 1000)],
                            tab_sh.at[pl.ds(s * 1000, 1000)])

        def zfill(i, _):
            fill_v[i, :] = jnp.zeros((D,), jnp.float32)
            return 0
        lax.fori_loop(0, NPT, zfill, 0)
        pltpu.sync_copy(fill_v.at[pl.ds(0, NPT)], acc_sh.at[pl.ds(s * NPT, NPT)])
        if with_cnt:
            pltpu.sync_copy(fill_v.at[pl.ds(0, NPT)],
                            cnt_sh.at[pl.ds(s * NPT, NPT)])

            def ofill(i, _):
                fill_v[i, :] = jnp.ones((D,), jnp.float32)
                return 0
            lax.fori_loop(0, K, ofill, 0)
        plsc.subcore_barrier()

        ebase = wid * EPW
        for j in range(NCH):
            off = ebase + j * K
            pltpu.sync_copy(src_hbm.at[pl.ds(off, K)], src_v)
            pltpu.sync_copy(dst_hbm.at[pl.ds(off, K)], dst_v)
            pltpu.async_copy(tab_sh.at[src_v], rows_v, sem).wait()
            pltpu.sync_copy(rows_v, acc_sh.at[dst_v], add=True)
            if with_cnt:
                pltpu.sync_copy(fill_v, cnt_sh.at[dst_v], add=True)
        plsc.subcore_barrier()

        rb = s * NPT
        ob = c * NP + rb
        pltpu.sync_copy(acc_sh.at[pl.ds(rb, NPT)], out_hbm.at[pl.ds(ob, NPT)])
        if with_cnt:
            pltpu.sync_copy(cnt_sh.at[pl.ds(rb, NPT)],
                            cnt_hbm.at[pl.ds(ob, NPT)])

    return functools.partial(
        pl.kernel, mesh=mesh, out_type=out_type, scratch_types=scratch_types,
        compiler_params=pltpu.CompilerParams(use_tc_tiling_on_sc=False),
    )(body)


_agg_cnt = _make_agg(with_cnt=True)
_agg = _make_agg(with_cnt=False)


def _mid_body(p_ref, c_ref, r_ref, wl_ref, wr_ref, b_ref, y_ref, rr_ref):
    agg = p_ref[0] + p_ref[1]
    cnt = c_ref[0] + c_ref[1]
    inv = 1.0 / jnp.maximum(cnt, 1.0)
    h = jnp.maximum(agg * inv + r_ref[...], 0.0)
    y_ref[...] = jnp.dot(h, wl_ref[...], preferred_element_type=jnp.float32)
    rr_ref[...] = (
        jnp.dot(h, wr_ref[...], preferred_element_type=jnp.float32) + b_ref[...]
    )


def _mid(p1, c1, r1, W2l, W2r, b2):
    return pl.pallas_call(
        _mid_body,
        grid=(N // RB,),
        in_specs=[
            pl.BlockSpec((NC, RB, D), lambda i: (0, i, 0)),
            pl.BlockSpec((NC, RB, D), lambda i: (0, i, 0)),
            pl.BlockSpec((RB, D), lambda i: (i, 0)),
            pl.BlockSpec((D, D), lambda i: (0, 0)),
            pl.BlockSpec((D, D), lambda i: (0, 0)),
            pl.BlockSpec((1, D), lambda i: (0, 0)),
        ],
        out_specs=[
            pl.BlockSpec((RB, D), lambda i: (i, 0)),
            pl.BlockSpec((RB, D), lambda i: (i, 0)),
        ],
        out_shape=[
            jax.ShapeDtypeStruct((N, D), jnp.float32),
            jax.ShapeDtypeStruct((N, D), jnp.float32),
        ],
    )(p1, c1, r1, W2l, W2r, b2.reshape(1, D))


def _fin_body(p_ref, c_ref, r_ref, o_ref):
    agg = p_ref[0] + p_ref[1]
    cnt = c_ref[0] + c_ref[1]
    inv = 1.0 / jnp.maximum(cnt, 1.0)
    z = agg * inv + r_ref[...]
    m = jnp.max(z, axis=1, keepdims=True)
    e = jnp.exp(z - m)
    o_ref[...] = e / jnp.sum(e, axis=1, keepdims=True)


def _fin(p2, c1, r2):
    return pl.pallas_call(
        _fin_body,
        grid=(N // RB,),
        in_specs=[
            pl.BlockSpec((NC, RB, D), lambda i: (0, i, 0)),
            pl.BlockSpec((NC, RB, D), lambda i: (0, i, 0)),
            pl.BlockSpec((RB, D), lambda i: (i, 0)),
        ],
        out_specs=pl.BlockSpec((RB, D), lambda i: (i, 0)),
        out_shape=jax.ShapeDtypeStruct((N, D), jnp.float32),
    )(p2, c1, r2)


def kernel(x, edge_index, W1l, b1, W1r, W2l, b2, W2r):
    src = edge_index[0].astype(jnp.int32)
    dst = edge_index[1].astype(jnp.int32)
    y1, r1 = _proj1(x, W1l, W1r, b1)
    p1, c1 = _agg_cnt(y1, src, dst)
    p1 = p1.reshape(NC, NP, D)[:, :N, :]
    c1 = c1.reshape(NC, NP, D)[:, :N, :]
    y2, r2 = _mid(p1, c1, r1, W2l, W2r, b2)
    (p2,) = _agg(y2, src, dst)
    p2 = p2.reshape(NC, NP, D)[:, :N, :]
    return _fin(p2, c1, r2)


# speedup vs baseline: 20.7284x; 1.0720x over previous
import functools

import jax
import jax.numpy as jnp
from jax import lax
from jax.experimental import pallas as pl
from jax.experimental.pallas import tpu as pltpu
from jax.experimental.pallas import tpu_sc as plsc

N = 10000
E = 320000
D_IN = 128
D = 16

NC = 2
NS = 16
NW = NC * NS
EPW = E // NW
K = 2000
NCH = EPW // K
NP = 10240
NPT = NP // NS

RB = 2000


def _proj1_body(x_ref, wl_ref, wr_ref, b_ref, y_ref, r_ref):
    xb = x_ref[...]
    y_ref[...] = jnp.dot(xb, wl_ref[...], preferred_element_type=jnp.float32)
    r_ref[...] = (
        jnp.dot(xb, wr_ref[...], preferred_element_type=jnp.float32) + b_ref[...]
    )


def _proj1(x, W1l, W1r, b1):
    return pl.pallas_call(
        _proj1_body,
        grid=(N // RB,),
        in_specs=[
            pl.BlockSpec((RB, D_IN), lambda i: (i, 0)),
            pl.BlockSpec((D_IN, D), lambda i: (0, 0)),
            pl.BlockSpec((D_IN, D), lambda i: (0, 0)),
            pl.BlockSpec((1, D), lambda i: (0, 0)),
        ],
        out_specs=[
            pl.BlockSpec((RB, D), lambda i: (i, 0)),
            pl.BlockSpec((RB, D), lambda i: (i, 0)),
        ],
        out_shape=[
            jax.ShapeDtypeStruct((N, D), jnp.float32),
            jax.ShapeDtypeStruct((N, D), jnp.float32),
        ],
    )(x, W1l, W1r, b1.reshape(1, D))


EPT = E // NS
HCH = EPT // K
NV = NPT // 16


def _make_agg(first_layer):
    mesh = plsc.VectorSubcoreMesh(core_axis_name="c", subcore_axis_name="s")
    out_type = [jax.ShapeDtypeStruct((NC * NP, D), jnp.float32)]
    if first_layer:
        out_type.append(jax.ShapeDtypeStruct((NP,), jnp.float32))
    scratch_types = [
        pltpu.VMEM((K,), jnp.int32),
        pltpu.VMEM((K,), jnp.int32),
        pltpu.VMEM((K, D), jnp.float32),
        pltpu.VMEM((NPT,), jnp.float32),
        pltpu.VMEM_SHARED((N, D), jnp.float32),
        pltpu.VMEM_SHARED((NP, D), jnp.float32),
    ]
    if first_layer:
        scratch_types += [
            pltpu.VMEM((NP,), jnp.float32),
            pltpu.VMEM((NS, NPT), jnp.float32),
            pltpu.VMEM_SHARED((NS, NP), jnp.float32),
        ]
    scratch_types.append(pltpu.SemaphoreType.DMA)

    def body(*args):
        if first_layer:
            (y_hbm, src_hbm, dst_hbm, out_hbm, inv_hbm,
             src_v, dst_v, rows_v, ivec_v, tab_sh, acc_sh,
             cnt_v, cbuf_v, cnt_sh, sem) = args
        else:
            (y_hbm, src_hbm, dst_hbm, inv_hbm, out_hbm,
             src_v, dst_v, rows_v, ivec_v, tab_sh, acc_sh, sem) = args
        c = lax.axis_index("c")
        s = lax.axis_index("s")
        wid = c * NS + s
        zeros16 = jnp.zeros((16,), jnp.float32)

        @pl.when(s < 10)
        def _stage():
            pltpu.sync_copy(y_hbm.at[pl.ds(s * 1000, 1000)],
                            tab_sh.at[pl.ds(s * 1000, 1000)])

        def zfill(i, _):
            rows_v[i, :] = zeros16
            return 0
        lax.fori_loop(0, NPT, zfill, 0)
        pltpu.sync_copy(rows_v.at[pl.ds(0, NPT)], acc_sh.at[pl.ds(s * NPT, NPT)])

        if first_layer:
            def hzero(i, _):
                cnt_v[pl.ds(i * 16, 16)] = zeros16
                return 0
            lax.fori_loop(0, NP // 16, hzero, 0)
            ones16 = jnp.ones((16,), jnp.float32)
            for jc in range(HCH):
                pltpu.sync_copy(dst_hbm.at[pl.ds(s * EPT + jc * K, K)], dst_v)

                def hbin(i, _):
                    idx = dst_v[pl.ds(i * 16, 16)]
                    plsc.addupdate_scatter(cnt_v, [idx], ones16)
                    return 0
                lax.fori_loop(0, K // 16, hbin, 0)
            pltpu.sync_copy(cnt_v, cnt_sh.at[s])
        else:
            pltpu.sync_copy(inv_hbm.at[pl.ds(s * NPT, NPT)], ivec_v)
        plsc.subcore_barrier()

        ebase = wid * EPW
        for j in range(NCH):
            off = ebase + j * K
            pltpu.sync_copy(src_hbm.at[pl.ds(off, K)], src_v)
            pltpu.sync_copy(dst_hbm.at[pl.ds(off, K)], dst_v)
            pltpu.async_copy(tab_sh.at[src_v], rows_v, sem).wait()
            pltpu.sync_copy(rows_v, acc_sh.at[dst_v], add=True)

        if first_layer:
            pltpu.sync_copy(cnt_sh.at[:, pl.ds(s * NPT, NPT)], cbuf_v)

            def hmerge(i, _):
                tot = cbuf_v[0, pl.ds(i * 16, 16)]
                for r in range(1, NS):
                    tot = tot + cbuf_v[r, pl.ds(i * 16, 16)]
                ivec_v[pl.ds(i * 16, 16)] = 1.0 / jnp.maximum(tot, 1.0)
                return 0
            lax.fori_loop(0, NV, hmerge, 0)
        plsc.subcore_barrier()

        rb = s * NPT
        pltpu.sync_copy(acc_sh.at[pl.ds(rb, NPT)], rows_v.at[pl.ds(0, NPT)])

        def scale(i, _):
            iv = ivec_v[pl.ds(i * 16, 16)]
            for l in range(16):
                r = i * 16 + l
                rows_v[r, :] = rows_v[r, :] * jnp.full((16,), iv[l],
                                                       jnp.float32)
            return 0
        lax.fori_loop(0, NV, scale, 0)
        pltpu.sync_copy(rows_v.at[pl.ds(0, NPT)],
                        out_hbm.at[pl.ds(c * NP + rb, NPT)])
        if first_layer:
            @pl.when(c == 0)
            def _winv():
                pltpu.sync_copy(ivec_v, inv_hbm.at[pl.ds(rb, NPT)])

    return functools.partial(
        pl.kernel, mesh=mesh, out_type=out_type, scratch_types=scratch_types,
        compiler_params=pltpu.CompilerParams(use_tc_tiling_on_sc=False,
                                             needs_layout_passes=False),
    )(body)


_agg1 = _make_agg(first_layer=True)
_agg2 = _make_agg(first_layer=False)


def _mid_body(p_ref, r_ref, wl_ref, wr_ref, b_ref, y_ref, rr_ref):
    h = jnp.maximum(p_ref[0] + p_ref[1] + r_ref[...], 0.0)
    y_ref[...] = jnp.dot(h, wl_ref[...], preferred_element_type=jnp.float32)
    rr_ref[...] = (
        jnp.dot(h, wr_ref[...], preferred_element_type=jnp.float32) + b_ref[...]
    )


def _mid(p1, r1, W2l, W2r, b2):
    return pl.pallas_call(
        _mid_body,
        grid=(N // RB,),
        in_specs=[
            pl.BlockSpec((NC, RB, D), lambda i: (0, i, 0)),
            pl.BlockSpec((RB, D), lambda i: (i, 0)),
            pl.BlockSpec((D, D), lambda i: (0, 0)),
            pl.BlockSpec((D, D), lambda i: (0, 0)),
            pl.BlockSpec((1, D), lambda i: (0, 0)),
        ],
        out_specs=[
            pl.BlockSpec((RB, D), lambda i: (i, 0)),
            pl.BlockSpec((RB, D), lambda i: (i, 0)),
        ],
        out_shape=[
            jax.ShapeDtypeStruct((N, D), jnp.float32),
            jax.ShapeDtypeStruct((N, D), jnp.float32),
        ],
    )(p1, r1, W2l, W2r, b2.reshape(1, D))


def _fin_body(p_ref, r_ref, o_ref):
    z = p_ref[0] + p_ref[1] + r_ref[...]
    m = jnp.max(z, axis=1, keepdims=True)
    e = jnp.exp(z - m)
    o_ref[...] = e / jnp.sum(e, axis=1, keepdims=True)


def _fin(p2, r2):
    return pl.pallas_call(
        _fin_body,
        grid=(N // RB,),
        in_specs=[
            pl.BlockSpec((NC, RB, D), lambda i: (0, i, 0)),
            pl.BlockSpec((RB, D), lambda i: (i, 0)),
        ],
        out_specs=pl.BlockSpec((RB, D), lambda i: (i, 0)),
        out_shape=jax.ShapeDtypeStruct((N, D), jnp.float32),
    )(p2, r2)


def kernel(x, edge_index, W1l, b1, W1r, W2l, b2, W2r):
    src = edge_index[0].astype(jnp.int32)
    dst = edge_index[1].astype(jnp.int32)
    y1, r1 = _proj1(x, W1l, W1r, b1)
    p1, inv = _agg1(y1, src, dst)
    p1 = p1.reshape(NC, NP, D)[:, :N, :]
    y2, r2 = _mid(p1, r1, W2l, W2r, b2)
    (p2,) = _agg2(y2, src, dst, inv)
    p2 = p2.reshape(NC, NP, D)[:, :N, :]
    return _fin(p2, r2)


# speedup vs baseline: 22.5587x; 1.0883x over previous
import functools

import jax
import jax.numpy as jnp
from jax import lax
from jax.experimental import pallas as pl
from jax.experimental.pallas import tpu as pltpu
from jax.experimental.pallas import tpu_sc as plsc

N = 10000
E = 320000
D_IN = 128
D = 16

NC = 2
NS = 16
NW = NC * NS
EPW = E // NW
K = 2000
NCH = EPW // K
NP = 10240
NPT = NP // NS

RB = 2000


def _proj1_body(x_ref, wl_ref, wr_ref, b_ref, y_ref, r_ref):
    xb = x_ref[...]
    y_ref[...] = jnp.dot(xb, wl_ref[...], preferred_element_type=jnp.float32)
    r_ref[...] = (
        jnp.dot(xb, wr_ref[...], preferred_element_type=jnp.float32) + b_ref[...]
    )


def _proj1(x, W1l, W1r, b1):
    return pl.pallas_call(
        _proj1_body,
        grid=(N // RB,),
        in_specs=[
            pl.BlockSpec((RB, D_IN), lambda i: (i, 0)),
            pl.BlockSpec((D_IN, D), lambda i: (0, 0)),
            pl.BlockSpec((D_IN, D), lambda i: (0, 0)),
            pl.BlockSpec((1, D), lambda i: (0, 0)),
        ],
        out_specs=[
            pl.BlockSpec((RB, D), lambda i: (i, 0)),
            pl.BlockSpec((RB, D), lambda i: (i, 0)),
        ],
        out_shape=[
            jax.ShapeDtypeStruct((N, D), jnp.float32),
            jax.ShapeDtypeStruct((N, D), jnp.float32),
        ],
    )(x, W1l, W1r, b1.reshape(1, D))


EPT = E // NS
HCH = EPT // K
NV = NPT // 16


def _make_agg(first_layer):
    mesh = plsc.VectorSubcoreMesh(core_axis_name="c", subcore_axis_name="s")
    out_type = [jax.ShapeDtypeStruct((NC * NP, D), jnp.float32)]
    if first_layer:
        out_type.append(jax.ShapeDtypeStruct((NP,), jnp.float32))
    scratch_types = [
        pltpu.VMEM((K,), jnp.int32),
        pltpu.VMEM((K,), jnp.int32),
        pltpu.VMEM((K, D), jnp.float32),
        pltpu.VMEM((K,), jnp.int32),
        pltpu.VMEM((K,), jnp.int32),
        pltpu.VMEM((K, D), jnp.float32),
        pltpu.VMEM((NPT,), jnp.float32),
        pltpu.VMEM_SHARED((N, D), jnp.float32),
        pltpu.VMEM_SHARED((NP, D), jnp.float32),
    ]
    if first_layer:
        scratch_types += [
            pltpu.VMEM((NP,), jnp.float32),
            pltpu.VMEM((NS, NPT), jnp.float32),
            pltpu.VMEM_SHARED((NS, NP), jnp.float32),
        ]
    scratch_types += [pltpu.SemaphoreType.DMA] * 6

    def body(*args):
        if first_layer:
            (y_hbm, src_hbm, dst_hbm, out_hbm, inv_hbm,
             src_a, dst_a, rows_a, src_b, dst_b, rows_b, ivec_v,
             tab_sh, acc_sh, cnt_v, cbuf_v, cnt_sh,
             gsem_a, gsem_b, ssem_a, ssem_b, stg_sem, zsem) = args
        else:
            (y_hbm, src_hbm, dst_hbm, inv_hbm, out_hbm,
             src_a, dst_a, rows_a, src_b, dst_b, rows_b, ivec_v,
             tab_sh, acc_sh,
             gsem_a, gsem_b, ssem_a, ssem_b, stg_sem, zsem) = args
        c = lax.axis_index("c")
        s = lax.axis_index("s")
        wid = c * NS + s
        zeros16 = jnp.zeros((16,), jnp.float32)

        @pl.when(s < 10)
        def _stage():
            pltpu.async_copy(y_hbm.at[pl.ds(s * 1000, 1000)],
                             tab_sh.at[pl.ds(s * 1000, 1000)], stg_sem)

        def zfill(i, _):
            rows_a[i, :] = zeros16
            return 0
        lax.fori_loop(0, NPT, zfill, 0)
        zh = pltpu.async_copy(rows_a.at[pl.ds(0, NPT)],
                              acc_sh.at[pl.ds(s * NPT, NPT)], gsem_a)

        if first_layer:
            def hzero(i, _):
                cnt_v[pl.ds(i * 16, 16)] = zeros16
                return 0
            lax.fori_loop(0, NP // 16, hzero, 0)
            ones16 = jnp.ones((16,), jnp.float32)
            hbufs = [dst_a, dst_b]
            hsems = [ssem_a, ssem_b]
            pltpu.async_copy(dst_hbm.at[pl.ds(s * EPT, K)], dst_a, ssem_a)
            hh = [pltpu.make_async_copy(dst_hbm.at[pl.ds(s * EPT, K)],
                                        dst_a, ssem_a), None]
            for jc in range(HCH):
                cu, nx = jc % 2, (jc + 1) % 2
                if jc + 1 < HCH:
                    off = s * EPT + (jc + 1) * K
                    pltpu.async_copy(dst_hbm.at[pl.ds(off, K)],
                                     hbufs[nx], hsems[nx])
                    hh[nx] = pltpu.make_async_copy(
                        dst_hbm.at[pl.ds(off, K)], hbufs[nx], hsems[nx])
                hh[cu].wait()
                hbuf = hbufs[cu]

                def hbin(i, _):
                    idx = hbuf[pl.ds(i * 16, 16)]
                    plsc.addupdate_scatter(cnt_v, [idx], ones16)
                    return 0
                lax.fori_loop(0, K // 16, hbin, 0)
            pltpu.sync_copy(cnt_v, cnt_sh.at[s])
        else:
            pltpu.sync_copy(inv_hbm.at[pl.ds(s * NPT, NPT)], ivec_v)

        zh.wait()
        @pl.when(s < 10)
        def _stagewait():
            pltpu.make_async_copy(y_hbm.at[pl.ds(s * 1000, 1000)],
                                  tab_sh.at[pl.ds(s * 1000, 1000)],
                                  stg_sem).wait()
        plsc.subcore_barrier()

        bufs = [(src_a, dst_a, rows_a, gsem_a, ssem_a),
                (src_b, dst_b, rows_b, gsem_b, ssem_b)]
        ebase = wid * EPW
        pltpu.sync_copy(src_hbm.at[pl.ds(ebase, K)], src_a)
        pltpu.sync_copy(dst_hbm.at[pl.ds(ebase, K)], dst_a)
        ghand = [None, None]
        shand = [None, None]
        ghand[0] = pltpu.async_copy(tab_sh.at[src_a], rows_a, gsem_a)
        for j in range(NCH):
            cur = j % 2
            nxt = (j + 1) % 2
            s_c, d_c, r_c, g_c, ss_c = bufs[cur]
            s_n, d_n, r_n, g_n, ss_n = bufs[nxt]
            if j + 1 < NCH:
                if shand[nxt] is not None:
                    shand[nxt].wait()
                    shand[nxt] = None
                off = ebase + (j + 1) * K
                pltpu.sync_copy(src_hbm.at[pl.ds(off, K)], s_n)
                pltpu.sync_copy(dst_hbm.at[pl.ds(off, K)], d_n)
            ghand[cur].wait()
            if j + 1 < NCH:
                ghand[nxt] = pltpu.async_copy(tab_sh.at[s_n], r_n, g_n)
            shand[cur] = pltpu.async_copy(r_c, acc_sh.at[d_c], ss_c,
                                          add=True)

        if first_layer:
            pltpu.sync_copy(cnt_sh.at[:, pl.ds(s * NPT, NPT)], cbuf_v)

            def hmerge(i, _):
                tot = cbuf_v[0, pl.ds(i * 16, 16)]
                for r in range(1, NS):
                    tot = tot + cbuf_v[r, pl.ds(i * 16, 16)]
                ivec_v[pl.ds(i * 16, 16)] = 1.0 / jnp.maximum(tot, 1.0)
                return 0
            lax.fori_loop(0, NV, hmerge, 0)
        for h in shand:
            if h is not None:
                h.wait()
        plsc.subcore_barrier()

        rb = s * NPT
        pltpu.sync_copy(acc_sh.at[pl.ds(rb, NPT)], rows_a.at[pl.ds(0, NPT)])

        def scale(i, _):
            iv = ivec_v[pl.ds(i * 16, 16)]
            for l in range(16):
                r = i * 16 + l
                rows_a[r, :] = rows_a[r, :] * jnp.full((16,), iv[l],
                                                       jnp.float32)
            return 0
        lax.fori_loop(0, NV, scale, 0)
        pltpu.sync_copy(rows_a.at[pl.ds(0, NPT)],
                        out_hbm.at[pl.ds(c * NP + rb, NPT)])
        if first_layer:
            @pl.when(c == 0)
            def _winv():
                pltpu.sync_copy(ivec_v, inv_hbm.at[pl.ds(rb, NPT)])

    return functools.partial(
        pl.kernel, mesh=mesh, out_type=out_type, scratch_types=scratch_types,
        compiler_params=pltpu.CompilerParams(use_tc_tiling_on_sc=False,
                                             needs_layout_passes=False),
    )(body)


_agg1 = _make_agg(first_layer=True)
_agg2 = _make_agg(first_layer=False)


def _mid_body(p_ref, r_ref, wl_ref, wr_ref, b_ref, y_ref, rr_ref):
    h = jnp.maximum(p_ref[0] + p_ref[1] + r_ref[...], 0.0)
    y_ref[...] = jnp.dot(h, wl_ref[...], preferred_element_type=jnp.float32)
    rr_ref[...] = (
        jnp.dot(h, wr_ref[...], preferred_element_type=jnp.float32) + b_ref[...]
    )


def _mid(p1, r1, W2l, W2r, b2):
    return pl.pallas_call(
        _mid_body,
        grid=(N // RB,),
        in_specs=[
            pl.BlockSpec((NC, RB, D), lambda i: (0, i, 0)),
            pl.BlockSpec((RB, D), lambda i: (i, 0)),
            pl.BlockSpec((D, D), lambda i: (0, 0)),
            pl.BlockSpec((D, D), lambda i: (0, 0)),
            pl.BlockSpec((1, D), lambda i: (0, 0)),
        ],
        out_specs=[
            pl.BlockSpec((RB, D), lambda i: (i, 0)),
            pl.BlockSpec((RB, D), lambda i: (i, 0)),
        ],
        out_shape=[
            jax.ShapeDtypeStruct((N, D), jnp.float32),
            jax.ShapeDtypeStruct((N, D), jnp.float32),
        ],
    )(p1, r1, W2l, W2r, b2.reshape(1, D))


def _fin_body(p_ref, r_ref, o_ref):
    z = p_ref[0] + p_ref[1] + r_ref[...]
    m = jnp.max(z, axis=1, keepdims=True)
    e = jnp.exp(z - m)
    o_ref[...] = e / jnp.sum(e, axis=1, keepdims=True)


def _fin(p2, r2):
    return pl.pallas_call(
        _fin_body,
        grid=(N // RB,),
        in_specs=[
            pl.BlockSpec((NC, RB, D), lambda i: (0, i, 0)),
            pl.BlockSpec((RB, D), lambda i: (i, 0)),
        ],
        out_specs=pl.BlockSpec((RB, D), lambda i: (i, 0)),
        out_shape=jax.ShapeDtypeStruct((N, D), jnp.float32),
    )(p2, r2)


def kernel(x, edge_index, W1l, b1, W1r, W2l, b2, W2r):
    src = edge_index[0].astype(jnp.int32)
    dst = edge_index[1].astype(jnp.int32)
    y1, r1 = _proj1(x, W1l, W1r, b1)
    p1, inv = _agg1(y1, src, dst)
    p1 = p1.reshape(NC, NP, D)[:, :N, :]
    y2, r2 = _mid(p1, r1, W2l, W2r, b2)
    (p2,) = _agg2(y2, src, dst, inv)
    p2 = p2.reshape(NC, NP, D)[:, :N, :]
    return _fin(p2, r2)


# speedup vs baseline: 25.4972x; 1.1303x over previous
import functools

import jax
import jax.numpy as jnp
from jax import lax
from jax.experimental import pallas as pl
from jax.experimental.pallas import tpu as pltpu
from jax.experimental.pallas import tpu_sc as plsc

N = 10000
E = 320000
D_IN = 128
D = 16

NC = 2
NS = 16
NW = NC * NS
EPW = E // NW
K = 1000
NCH = EPW // K
EPT = E // NS
HCH = EPT // K
NP = 10240
NPT = NP // NS
NV = NPT // 16

RB = NP // 8
BR8 = NP // 8 // 8


def _proj1_body(x_ref, wl_ref, wr_ref, b_ref, y_ref, r_ref):
    xb = x_ref[...]
    y_ref[...] = jnp.dot(xb, wl_ref[...], preferred_element_type=jnp.float32)
    r_ref[...] = (
        jnp.dot(xb, wr_ref[...], preferred_element_type=jnp.float32) + b_ref[...]
    )


def _proj1(x, W1l, W1r, b1):
    return pl.pallas_call(
        _proj1_body,
        grid=(8,),
        in_specs=[
            pl.BlockSpec((RB, D_IN), lambda i: (i, 0)),
            pl.BlockSpec((D_IN, D), lambda i: (0, 0)),
            pl.BlockSpec((D_IN, D), lambda i: (0, 0)),
            pl.BlockSpec((1, D), lambda i: (0, 0)),
        ],
        out_specs=[
            pl.BlockSpec((RB, D), lambda i: (i, 0)),
            pl.BlockSpec((RB, D), lambda i: (i, 0)),
        ],
        out_shape=[
            jax.ShapeDtypeStruct((NP, D), jnp.float32),
            jax.ShapeDtypeStruct((NP, D), jnp.float32),
        ],
    )(x, W1l, W1r, b1.reshape(1, D))


def _make_agg(first_layer):
    mesh = plsc.VectorSubcoreMesh(core_axis_name="c", subcore_axis_name="s")
    out_type = [jax.ShapeDtypeStruct((NC * NP, D), jnp.float32)]
    if first_layer:
        out_type.append(jax.ShapeDtypeStruct((NP,), jnp.float32))
    scratch_types = [
        pltpu.VMEM((K,), jnp.int32),
        pltpu.VMEM((K,), jnp.int32),
        pltpu.VMEM((K, D), jnp.float32),
        pltpu.VMEM((K,), jnp.int32),
        pltpu.VMEM((K,), jnp.int32),
        pltpu.VMEM((K, D), jnp.float32),
        pltpu.VMEM((NPT,), jnp.float32),
        pltpu.VMEM((NPT, D), jnp.float32),
        pltpu.VMEM_SHARED((N, D), jnp.float32),
        pltpu.VMEM_SHARED((NP, D), jnp.float32),
    ]
    if first_layer:
        scratch_types += [
            pltpu.VMEM((NP,), jnp.float32),
            pltpu.VMEM((NS, NPT), jnp.float32),
            pltpu.VMEM_SHARED((NS, NP), jnp.float32),
        ]
    scratch_types += [pltpu.SemaphoreType.DMA] * 6

    def body(*args):
        if first_layer:
            (y_hbm, src_hbm, dst_hbm, r_hbm, out_hbm, inv_hbm,
             src_a, dst_a, rows_a, src_b, dst_b, rows_b, ivec_v, rbuf_v,
             tab_sh, acc_sh, cnt_v, cbuf_v, cnt_sh,
             gsem_a, gsem_b, ssem_a, ssem_b, stg_sem, zsem) = args
        else:
            (y_hbm, src_hbm, dst_hbm, inv_hbm, r_hbm, out_hbm,
             src_a, dst_a, rows_a, src_b, dst_b, rows_b, ivec_v, rbuf_v,
             tab_sh, acc_sh,
             gsem_a, gsem_b, ssem_a, ssem_b, stg_sem, zsem) = args
        c = lax.axis_index("c")
        s = lax.axis_index("s")
        wid = c * NS + s
        zeros16 = jnp.zeros((16,), jnp.float32)

        @pl.when(s < 10)
        def _stage():
            pltpu.async_copy(y_hbm.at[pl.ds(s * 1000, 1000)],
                             tab_sh.at[pl.ds(s * 1000, 1000)], stg_sem)

        def zfill(i, _):
            rows_a[i, :] = zeros16
            return 0
        lax.fori_loop(0, NPT, zfill, 0)
        zh = pltpu.async_copy(rows_a.at[pl.ds(0, NPT)],
                              acc_sh.at[pl.ds(s * NPT, NPT)], gsem_a)

        if first_layer:
            def hzero(i, _):
                cnt_v[pl.ds(i * 16, 16)] = zeros16
                return 0
            lax.fori_loop(0, NP // 16, hzero, 0)
            ones16 = jnp.ones((16,), jnp.float32)
            hbufs = [dst_a, dst_b]
            hsems = [ssem_a, ssem_b]
            pltpu.async_copy(dst_hbm.at[pl.ds(s * EPT, K)], dst_a, ssem_a)
            hh = [pltpu.make_async_copy(dst_hbm.at[pl.ds(s * EPT, K)],
                                        dst_a, ssem_a), None]
            for jc in range(HCH):
                cu, nx = jc % 2, (jc + 1) % 2
                if jc + 1 < HCH:
                    off = s * EPT + (jc + 1) * K
                    pltpu.async_copy(dst_hbm.at[pl.ds(off, K)],
                                     hbufs[nx], hsems[nx])
                    hh[nx] = pltpu.make_async_copy(
                        dst_hbm.at[pl.ds(off, K)], hbufs[nx], hsems[nx])
                hh[cu].wait()
                hbuf = hbufs[cu]

                def hbin(i, _):
                    idx = hbuf[pl.ds(i * 16, 16)]
                    plsc.addupdate_scatter(cnt_v, [idx], ones16)
                    return 0
                lax.fori_loop(0, K // 16, hbin, 0)
            pltpu.sync_copy(cnt_v, cnt_sh.at[s])
        else:
            pltpu.sync_copy(inv_hbm.at[pl.ds(s * NPT, NPT)], ivec_v)

        zh.wait()
        @pl.when(s < 10)
        def _stagewait():
            pltpu.make_async_copy(y_hbm.at[pl.ds(s * 1000, 1000)],
                                  tab_sh.at[pl.ds(s * 1000, 1000)],
                                  stg_sem).wait()
        plsc.subcore_barrier()

        bufs = [(src_a, dst_a, rows_a, gsem_a, ssem_a),
                (src_b, dst_b, rows_b, gsem_b, ssem_b)]
        ebase = wid * EPW
        pltpu.sync_copy(src_hbm.at[pl.ds(ebase, K)], src_a)
        pltpu.sync_copy(dst_hbm.at[pl.ds(ebase, K)], dst_a)
        ghand = [None, None]
        shand = [None, None]
        ghand[0] = pltpu.async_copy(tab_sh.at[src_a], rows_a, gsem_a)
        for j in range(NCH):
            cur = j % 2
            nxt = (j + 1) % 2
            s_c, d_c, r_c, g_c, ss_c = bufs[cur]
            s_n, d_n, r_n, g_n, ss_n = bufs[nxt]
            if j + 1 < NCH:
                if shand[nxt] is not None:
                    shand[nxt].wait()
                    shand[nxt] = None
                off = ebase + (j + 1) * K
                pltpu.sync_copy(src_hbm.at[pl.ds(off, K)], s_n)
                pltpu.sync_copy(dst_hbm.at[pl.ds(off, K)], d_n)
            ghand[cur].wait()
            if j + 1 < NCH:
                ghand[nxt] = pltpu.async_copy(tab_sh.at[s_n], r_n, g_n)
            shand[cur] = pltpu.async_copy(r_c, acc_sh.at[d_c], ss_c,
                                          add=True)

        rh = pltpu.async_copy(r_hbm.at[pl.ds(s * NPT, NPT)], rbuf_v, zsem)
        if first_layer:
            pltpu.sync_copy(cnt_sh.at[:, pl.ds(s * NPT, NPT)], cbuf_v)

            def hmerge(i, _):
                tot = cbuf_v[0, pl.ds(i * 16, 16)]
                for r in range(1, NS):
                    tot = tot + cbuf_v[r, pl.ds(i * 16, 16)]
                ivec_v[pl.ds(i * 16, 16)] = 1.0 / jnp.maximum(tot, 1.0)
                return 0
            lax.fori_loop(0, NV, hmerge, 0)
        for h in shand:
            if h is not None:
                h.wait()
        rh.wait()
        plsc.subcore_barrier()

        rb = s * NPT
        pltpu.sync_copy(acc_sh.at[pl.ds(rb, NPT)], rows_a.at[pl.ds(0, NPT)])
        rmask = jnp.full((16,), jnp.where(c == 0, 1.0, 0.0), jnp.float32)

        def scale(i, _):
            iv = ivec_v[pl.ds(i * 16, 16)]
            for l in range(16):
                r = i * 16 + l
                rows_a[r, :] = (rows_a[r, :] * jnp.full((16,), iv[l],
                                                        jnp.float32)
                                + rbuf_v[r, :] * rmask)
            return 0
        lax.fori_loop(0, NV, scale, 0)
        pltpu.sync_copy(rows_a.at[pl.ds(0, NPT)],
                        out_hbm.at[pl.ds(c * NP + rb, NPT)])
        if first_layer:
            @pl.when(c == 0)
            def _winv():
                pltpu.sync_copy(ivec_v, inv_hbm.at[pl.ds(rb, NPT)])

    return functools.partial(
        pl.kernel, mesh=mesh, out_type=out_type, scratch_types=scratch_types,
        compiler_params=pltpu.CompilerParams(use_tc_tiling_on_sc=False,
                                             needs_layout_passes=False),
    )(body)


_agg1 = _make_agg(first_layer=True)
_agg2 = _make_agg(first_layer=False)


def _mid_body(pa_ref, pb_ref, wl_ref, wr_ref, b_ref, y_ref, rr_ref):
    h8 = jnp.maximum(pa_ref[...] + pb_ref[...], 0.0)
    y_ref[...] = jnp.dot(h8, wl_ref[...], preferred_element_type=jnp.float32)
    rr_ref[...] = (
        jnp.dot(h8, wr_ref[...], preferred_element_type=jnp.float32) + b_ref[...]
    )


def _mid(p8, W2l8, W2r8, b2_8):
    return pl.pallas_call(
        _mid_body,
        grid=(8,),
        in_specs=[
            pl.BlockSpec((BR8, 128), lambda i: (i, 0)),
            pl.BlockSpec((BR8, 128), lambda i: (i + 8, 0)),
            pl.BlockSpec((128, 128), lambda i: (0, 0)),
            pl.BlockSpec((128, 128), lambda i: (0, 0)),
            pl.BlockSpec((1, 128), lambda i: (0, 0)),
        ],
        out_specs=[
            pl.BlockSpec((BR8, 128), lambda i: (i, 0)),
            pl.BlockSpec((BR8, 128), lambda i: (i, 0)),
        ],
        out_shape=[
            jax.ShapeDtypeStruct((NP // 8, 128), jnp.float32),
            jax.ShapeDtypeStruct((NP // 8, 128), jnp.float32),
        ],
    )(p8, p8, W2l8, W2r8, b2_8.reshape(1, 128))


def _fin_body(qa_ref, qb_ref, ones_ref, o_ref):
    z8 = qa_ref[...] + qb_ref[...]
    m = jnp.max(z8, axis=1, keepdims=True)
    e8 = jnp.exp(z8 - m)
    s8 = jnp.dot(e8, ones_ref[...], preferred_element_type=jnp.float32)
    o_ref[...] = e8 / s8


def _fin(q8, ones8):
    return pl.pallas_call(
        _fin_body,
        grid=(8,),
        in_specs=[
            pl.BlockSpec((BR8, 128), lambda i: (i, 0)),
            pl.BlockSpec((BR8, 128), lambda i: (i + 8, 0)),
            pl.BlockSpec((128, 128), lambda i: (0, 0)),
        ],
        out_specs=pl.BlockSpec((BR8, 128), lambda i: (i, 0)),
        out_shape=jax.ShapeDtypeStruct((NP // 8, 128), jnp.float32),
    )(q8, q8, ones8)


def _block_diag8(w):
    z = jnp.zeros((8, D, 8, D), jnp.float32)
    idx = jnp.arange(8)
    z = z.at[idx, :, idx, :].set(w)
    return z.reshape(8 * D, 8 * D)


def kernel(x, edge_index, W1l, b1, W1r, W2l, b2, W2r):
    src = edge_index[0].astype(jnp.int32)
    dst = edge_index[1].astype(jnp.int32)
    W2l8 = _block_diag8(W2l)
    W2r8 = _block_diag8(W2r)
    b2_8 = jnp.tile(b2, 8)
    ones8 = _block_diag8(jnp.ones((D, D), jnp.float32))
    y1, r1 = _proj1(x, W1l, W1r, b1)
    p1, inv = _agg1(y1, src, dst, r1)
    y2_8, r2_8 = _mid(p1.reshape(NC * NP // 8, 128), W2l8, W2r8, b2_8)
    (q,) = _agg2(y2_8.reshape(NP, D), src, dst, inv, r2_8.reshape(NP, D))
    out8 = _fin(q.reshape(NC * NP // 8, 128), ones8)
    return out8.reshape(NP, D)[:N]
